# initial kernel scaffold (unmeasured)
import jax
import jax.numpy as jnp
from jax import lax
from jax.experimental import pallas as pl
from jax.experimental.pallas import tpu as pltpu

N_DEV = 4
BM = 1024
CW = 2048
MESH = pl.DeviceIdType.MESH


def kernel(x, w_mat, scale_x, scale_w):
    partial = jnp.dot(
        x.astype(jnp.bfloat16),
        w_mat.astype(jnp.bfloat16),
        preferred_element_type=jnp.float32,
    )
    scale = (scale_x * scale_w).astype(jnp.float32)
    return _ring_allreduce(partial, scale)


def _ring_allreduce(partial, scale):
    m, n = partial.shape
    nc = n // CW

    def body(scale_ref, p_ref, out_ref, rs_recv, w_ref, va, vb,
             rs_send_sems, rs_recv_sems, ag_send_sems, ag_recv_sems,
             cp_sems):
        my = lax.axis_index("i")
        left = lax.rem(my + (N_DEV - 1), N_DEV)
        right = lax.rem(my + 1, N_DEV)

        barrier = pltpu.get_barrier_semaphore()
        for nbr in (left, right):
            pl.semaphore_signal(barrier, inc=1, device_id=(nbr,),
                                device_id_type=MESH)
        pl.semaphore_wait(barrier, 2)

        def rows(b):
            return pl.ds(b * BM, BM)

        def add_into(dst_fn, a_fn, b_fn, do_scale):
            for c in range(nc):
                cols = pl.ds(c * CW, CW)
                cpa = pltpu.make_async_copy(a_fn(cols), va, cp_sems.at[0])
                cpb = pltpu.make_async_copy(b_fn(cols), vb, cp_sems.at[1])
                cpa.start()
                cpb.start()
                cpa.wait()
                cpb.wait()
                if do_scale:
                    va[...] = (va[...] + vb[...]) * scale_ref[0]
                else:
                    va[...] = va[...] + vb[...]
                cpo = pltpu.make_async_copy(va, dst_fn(cols), cp_sems.at[2])
                cpo.start()
                cpo.wait()

        for s in range(3):
            src = p_ref.at[rows(my), :] if s == 0 else w_ref
            rdma = pltpu.make_async_remote_copy(
                src_ref=src,
                dst_ref=rs_recv.at[s],
                send_sem=rs_send_sems.at[s],
                recv_sem=rs_recv_sems.at[s],
                device_id=(right,),
                device_id_type=MESH,
            )
            rdma.start()
            rdma.wait()
            b = lax.rem(my + (2 * N_DEV - 1 - s), N_DEV)
            if s < 2:
                add_into(lambda cols: w_ref.at[:, cols],
                         lambda cols, b=b: p_ref.at[rows(b), cols],
                         lambda cols, s=s: rs_recv.at[s, :, cols],
                         do_scale=False)
            else:
                add_into(lambda cols, b=b: out_ref.at[rows(b), cols],
                         lambda cols, b=b: p_ref.at[rows(b), cols],
                         lambda cols, s=s: rs_recv.at[s, :, cols],
                         do_scale=True)

        for s in range(3):
            sb = lax.rem(my + (N_DEV + 1 - s), N_DEV)
            rdma = pltpu.make_async_remote_copy(
                src_ref=out_ref.at[rows(sb), :],
                dst_ref=out_ref.at[rows(sb), :],
                send_sem=ag_send_sems.at[s],
                recv_sem=ag_recv_sems.at[s],
                device_id=(right,),
                device_id_type=MESH,
            )
            rdma.start()
            rdma.wait()

    return pl.pallas_call(
        body,
        out_shape=jax.ShapeDtypeStruct((m, n), jnp.float32),
        in_specs=[
            pl.BlockSpec(memory_space=pltpu.MemorySpace.SMEM),
            pl.BlockSpec(memory_space=pl.ANY),
        ],
        out_specs=pl.BlockSpec(memory_space=pl.ANY),
        scratch_shapes=[
            pltpu.MemorySpace.HBM((3, BM, n), jnp.float32),
            pltpu.MemorySpace.HBM((BM, n), jnp.float32),
            pltpu.VMEM((BM, CW), jnp.float32),
            pltpu.VMEM((BM, CW), jnp.float32),
            pltpu.SemaphoreType.DMA((3,)),
            pltpu.SemaphoreType.DMA((3,)),
            pltpu.SemaphoreType.DMA((3,)),
            pltpu.SemaphoreType.DMA((3,)),
            pltpu.SemaphoreType.DMA((3,)),
        ],
        compiler_params=pltpu.CompilerParams(collective_id=0),
    )(scale, partial)


# baseline (device time: 2468802 ns/iter reference)
import jax
import jax.numpy as jnp
from jax import lax
from jax.experimental import pallas as pl
from jax.experimental.pallas import tpu as pltpu

N_DEV = 4
BM = 1024
CW = 2048
MESH = pl.DeviceIdType.MESH


def kernel(x, w_mat, scale_x, scale_w):
    partial = jnp.dot(
        x.astype(jnp.bfloat16),
        w_mat.astype(jnp.bfloat16),
        preferred_element_type=jnp.float32,
    )
    scale = (scale_x * scale_w).astype(jnp.float32)
    return _ring_allreduce(partial, scale)


def _ring_allreduce(partial, scale):
    m, n = partial.shape
    nc = n // CW

    def body(scale_ref, p_ref, out_ref, rs_recv, w_ref, va, vb,
             rs_send_sems, rs_recv_sems, ag_send_sems, ag_recv_sems,
             cp_sems):
        my = lax.axis_index("i")
        left = lax.rem(my + (N_DEV - 1), N_DEV)
        right = lax.rem(my + 1, N_DEV)

        barrier = pltpu.get_barrier_semaphore()
        for nbr in (left, right):
            pl.semaphore_signal(barrier, inc=1, device_id=(nbr,),
                                device_id_type=MESH)
        pl.semaphore_wait(barrier, 2)

        def rows(b):
            return pl.ds(b * BM, BM)

        def add_into(dst_fn, a_fn, b_fn, do_scale):
            for c in range(nc):
                cols = pl.ds(c * CW, CW)
                cpa = pltpu.make_async_copy(a_fn(cols), va, cp_sems.at[0])
                cpb = pltpu.make_async_copy(b_fn(cols), vb, cp_sems.at[1])
                cpa.start()
                cpb.start()
                cpa.wait()
                cpb.wait()
                if do_scale:
                    va[...] = (va[...] + vb[...]) * scale_ref[0]
                else:
                    va[...] = va[...] + vb[...]
                cpo = pltpu.make_async_copy(va, dst_fn(cols), cp_sems.at[2])
                cpo.start()
                cpo.wait()

        for s in range(3):
            src = p_ref.at[rows(my), :] if s == 0 else w_ref
            rdma = pltpu.make_async_remote_copy(
                src_ref=src,
                dst_ref=rs_recv.at[s],
                send_sem=rs_send_sems.at[s],
                recv_sem=rs_recv_sems.at[s],
                device_id=(right,),
                device_id_type=MESH,
            )
            rdma.start()
            rdma.wait()
            b = lax.rem(my + (2 * N_DEV - 1 - s), N_DEV)
            if s < 2:
                add_into(lambda cols: w_ref.at[:, cols],
                         lambda cols, b=b: p_ref.at[rows(b), cols],
                         lambda cols, s=s: rs_recv.at[s, :, cols],
                         do_scale=False)
            else:
                add_into(lambda cols, b=b: out_ref.at[rows(b), cols],
                         lambda cols, b=b: p_ref.at[rows(b), cols],
                         lambda cols, s=s: rs_recv.at[s, :, cols],
                         do_scale=True)

        for s in range(3):
            sb = lax.rem(my + (N_DEV + 1 - s), N_DEV)
            rdma = pltpu.make_async_remote_copy(
                src_ref=out_ref.at[rows(sb), :],
                dst_ref=out_ref.at[rows(sb), :],
                send_sem=ag_send_sems.at[s],
                recv_sem=ag_recv_sems.at[s],
                device_id=(right,),
                device_id_type=MESH,
            )
            rdma.start()
            rdma.wait()

    out, _, _ = pl.pallas_call(
        body,
        out_shape=[
            jax.ShapeDtypeStruct((m, n), jnp.float32),
            jax.ShapeDtypeStruct((3, BM, n), jnp.float32),
            jax.ShapeDtypeStruct((BM, n), jnp.float32),
        ],
        in_specs=[
            pl.BlockSpec(memory_space=pltpu.MemorySpace.SMEM),
            pl.BlockSpec(memory_space=pl.ANY),
        ],
        out_specs=[
            pl.BlockSpec(memory_space=pl.ANY),
            pl.BlockSpec(memory_space=pl.ANY),
            pl.BlockSpec(memory_space=pl.ANY),
        ],
        scratch_shapes=[
            pltpu.VMEM((BM, CW), jnp.float32),
            pltpu.VMEM((BM, CW), jnp.float32),
            pltpu.SemaphoreType.DMA((3,)),
            pltpu.SemaphoreType.DMA((3,)),
            pltpu.SemaphoreType.DMA((3,)),
            pltpu.SemaphoreType.DMA((3,)),
            pltpu.SemaphoreType.DMA((3,)),
        ],
        compiler_params=pltpu.CompilerParams(collective_id=0),
    )(scale, partial)
    return out


# device time: 1330347 ns/iter; 1.8558x vs baseline; 1.8558x over previous
import jax
import jax.numpy as jnp
from jax import lax
from jax.experimental import pallas as pl
from jax.experimental.pallas import tpu as pltpu

N_DEV = 4
BM = 1024
MESH = pl.DeviceIdType.MESH


def kernel(x, w_mat, scale_x, scale_w):
    partial = jnp.dot(
        x.astype(jnp.bfloat16),
        w_mat.astype(jnp.bfloat16),
        preferred_element_type=jnp.float32,
    )
    scale = (scale_x * scale_w).astype(jnp.float32)
    return _ring_allreduce(partial, scale)


def _ring_allreduce(partial, scale):
    m, n = partial.shape
    hl = n // 2
    sc = hl // 2

    def body(scale_ref, p_ref, out_ref, rs_recv, w_ref, va, vb,
             rs_send_sems, rs_recv_sems, ag_send_sems, ag_recv_sems,
             cp_sems):
        my = lax.axis_index("i")
        left = lax.rem(my + (N_DEV - 1), N_DEV)
        right = lax.rem(my + 1, N_DEV)

        barrier = pltpu.get_barrier_semaphore()
        for nbr in (left, right):
            pl.semaphore_signal(barrier, inc=1, device_id=(nbr,),
                                device_id_type=MESH)
        pl.semaphore_wait(barrier, 2)

        def rows(b):
            return pl.ds(b * BM, BM)

        def cols(d, c):
            return pl.ds(d * hl + c * sc, sc)

        def add_chunk(dst, a, bsrc, do_scale):
            cpa = pltpu.make_async_copy(a, va, cp_sems.at[0])
            cpb = pltpu.make_async_copy(bsrc, vb, cp_sems.at[1])
            cpa.start()
            cpb.start()
            cpa.wait()
            cpb.wait()
            if do_scale:
                va[...] = (va[...] + vb[...]) * scale_ref[0]
            else:
                va[...] = va[...] + vb[...]
            cpo = pltpu.make_async_copy(va, dst, cp_sems.at[2])
            cpo.start()
            cpo.wait()

        for s in range(3):
            rdmas = {}
            for d, tgt in ((0, right), (1, left)):
                for c in range(2):
                    src = (p_ref.at[rows(my), cols(d, c)] if s == 0
                           else w_ref.at[:, cols(d, c)])
                    k = s * 4 + d * 2 + c
                    r = pltpu.make_async_remote_copy(
                        src_ref=src,
                        dst_ref=rs_recv.at[s, :, cols(d, c)],
                        send_sem=rs_send_sems.at[k],
                        recv_sem=rs_recv_sems.at[k],
                        device_id=(tgt,),
                        device_id_type=MESH,
                    )
                    r.start()
                    rdmas[(d, c)] = r
            bA = lax.rem(my + (2 * N_DEV - 1 - s), N_DEV)
            bB = lax.rem(my + (1 + s), N_DEV)
            for d, c in ((0, 0), (1, 0), (0, 1), (1, 1)):
                rdmas[(d, c)].wait()
                b = bA if d == 0 else bB
                if s < 2:
                    add_chunk(w_ref.at[:, cols(d, c)],
                              p_ref.at[rows(b), cols(d, c)],
                              rs_recv.at[s, :, cols(d, c)],
                              do_scale=False)
                else:
                    add_chunk(out_ref.at[rows(b), cols(d, c)],
                              p_ref.at[rows(b), cols(d, c)],
                              rs_recv.at[s, :, cols(d, c)],
                              do_scale=True)

        for s in range(3):
            rdmas = []
            for d, tgt in ((0, right), (1, left)):
                if d == 0:
                    sb = lax.rem(my + (N_DEV + 1 - s), N_DEV)
                else:
                    sb = lax.rem(my + (N_DEV - 1 + s), N_DEV)
                half = pl.ds(d * hl, hl)
                r = pltpu.make_async_remote_copy(
                    src_ref=out_ref.at[rows(sb), half],
                    dst_ref=out_ref.at[rows(sb), half],
                    send_sem=ag_send_sems.at[s * 2 + d],
                    recv_sem=ag_recv_sems.at[s * 2 + d],
                    device_id=(tgt,),
                    device_id_type=MESH,
                )
                r.start()
                rdmas.append(r)
            for r in rdmas:
                r.wait()

    out, _, _ = pl.pallas_call(
        body,
        out_shape=[
            jax.ShapeDtypeStruct((m, n), jnp.float32),
            jax.ShapeDtypeStruct((3, BM, n), jnp.float32),
            jax.ShapeDtypeStruct((BM, n), jnp.float32),
        ],
        in_specs=[
            pl.BlockSpec(memory_space=pltpu.MemorySpace.SMEM),
            pl.BlockSpec(memory_space=pl.ANY),
        ],
        out_specs=[
            pl.BlockSpec(memory_space=pl.ANY),
            pl.BlockSpec(memory_space=pl.ANY),
            pl.BlockSpec(memory_space=pl.ANY),
        ],
        scratch_shapes=[
            pltpu.VMEM((BM, sc), jnp.float32),
            pltpu.VMEM((BM, sc), jnp.float32),
            pltpu.SemaphoreType.DMA((12,)),
            pltpu.SemaphoreType.DMA((12,)),
            pltpu.SemaphoreType.DMA((6,)),
            pltpu.SemaphoreType.DMA((6,)),
            pltpu.SemaphoreType.DMA((3,)),
        ],
        compiler_params=pltpu.CompilerParams(collective_id=0),
    )(scale, partial)
    return out


# device time: 1287317 ns/iter; 1.9178x vs baseline; 1.0334x over previous
import jax
import jax.numpy as jnp
from jax import lax
from jax.experimental import pallas as pl
from jax.experimental.pallas import tpu as pltpu

N_DEV = 4
BM = 1024
MESH = pl.DeviceIdType.MESH


def kernel(x, w_mat, scale_x, scale_w):
    partial = jnp.dot(
        x.astype(jnp.bfloat16),
        w_mat.astype(jnp.bfloat16),
        preferred_element_type=jnp.float32,
    )
    scale = (scale_x * scale_w).astype(jnp.float32)
    return _ring_allreduce(partial, scale)


def _ring_allreduce(partial, scale):
    m, n = partial.shape
    hl = n // 2
    sc = hl // 2

    def body(scale_ref, p_ref, out_ref, rs_recv, w_ref, va, vb,
             rs_send_sems, rs_recv_sems, ag_send_sems, ag_recv_sems,
             cp_sems):
        my = lax.axis_index("i")
        left = lax.rem(my + (N_DEV - 1), N_DEV)
        right = lax.rem(my + 1, N_DEV)

        barrier = pltpu.get_barrier_semaphore()
        for nbr in (left, right):
            pl.semaphore_signal(barrier, inc=1, device_id=(nbr,),
                                device_id_type=MESH)
        pl.semaphore_wait(barrier, 2)

        def rows(b):
            return pl.ds(b * BM, BM)

        def cols(d, c):
            return pl.ds(d * hl + c * sc, sc)

        def add_chunk(dst, a, bsrc, do_scale):
            cpa = pltpu.make_async_copy(a, va, cp_sems.at[0])
            cpb = pltpu.make_async_copy(bsrc, vb, cp_sems.at[1])
            cpa.start()
            cpb.start()
            cpa.wait()
            cpb.wait()
            if do_scale:
                va[...] = (va[...] + vb[...]) * scale_ref[0]
            else:
                va[...] = va[...] + vb[...]
            cpo = pltpu.make_async_copy(va, dst, cp_sems.at[2])
            cpo.start()
            cpo.wait()

        def rs_start(s, d, c):
            tgt = right if d == 0 else left
            src = (p_ref.at[rows(my), cols(d, c)] if s == 0
                   else w_ref.at[:, cols(d, c)])
            k = s * 4 + d * 2 + c
            r = pltpu.make_async_remote_copy(
                src_ref=src,
                dst_ref=rs_recv.at[s, :, cols(d, c)],
                send_sem=rs_send_sems.at[k],
                recv_sem=rs_recv_sems.at[k],
                device_id=(tgt,),
                device_id_type=MESH,
            )
            r.start()
            return r

        rdmas = {}
        for d in (0, 1):
            for c in (0, 1):
                rdmas[(0, d, c)] = rs_start(0, d, c)
        for s in range(3):
            bA = lax.rem(my + (2 * N_DEV - 1 - s), N_DEV)
            bB = lax.rem(my + (1 + s), N_DEV)
            for c, d in ((0, 0), (0, 1), (1, 0), (1, 1)):
                rdmas[(s, d, c)].wait()
                b = bA if d == 0 else bB
                if s < 2:
                    add_chunk(w_ref.at[:, cols(d, c)],
                              p_ref.at[rows(b), cols(d, c)],
                              rs_recv.at[s, :, cols(d, c)],
                              do_scale=False)
                    rdmas[(s + 1, d, c)] = rs_start(s + 1, d, c)
                else:
                    add_chunk(out_ref.at[rows(b), cols(d, c)],
                              p_ref.at[rows(b), cols(d, c)],
                              rs_recv.at[s, :, cols(d, c)],
                              do_scale=True)

        def ag_start(s, d):
            tgt = right if d == 0 else left
            if d == 0:
                sb = lax.rem(my + (N_DEV + 1 - s), N_DEV)
            else:
                sb = lax.rem(my + (N_DEV - 1 + s), N_DEV)
            half = pl.ds(d * hl, hl)
            r = pltpu.make_async_remote_copy(
                src_ref=out_ref.at[rows(sb), half],
                dst_ref=out_ref.at[rows(sb), half],
                send_sem=ag_send_sems.at[s * 2 + d],
                recv_sem=ag_recv_sems.at[s * 2 + d],
                device_id=(tgt,),
                device_id_type=MESH,
            )
            r.start()
            return r

        ag = {(0, 0): ag_start(0, 0), (0, 1): ag_start(0, 1)}
        for s in range(3):
            for d in (0, 1):
                ag[(s, d)].wait()
                if s < 2:
                    ag[(s + 1, d)] = ag_start(s + 1, d)

    out, _, _ = pl.pallas_call(
        body,
        out_shape=[
            jax.ShapeDtypeStruct((m, n), jnp.float32),
            jax.ShapeDtypeStruct((3, BM, n), jnp.float32),
            jax.ShapeDtypeStruct((BM, n), jnp.float32),
        ],
        in_specs=[
            pl.BlockSpec(memory_space=pltpu.MemorySpace.SMEM),
            pl.BlockSpec(memory_space=pl.ANY),
        ],
        out_specs=[
            pl.BlockSpec(memory_space=pl.ANY),
            pl.BlockSpec(memory_space=pl.ANY),
            pl.BlockSpec(memory_space=pl.ANY),
        ],
        scratch_shapes=[
            pltpu.VMEM((BM, sc), jnp.float32),
            pltpu.VMEM((BM, sc), jnp.float32),
            pltpu.SemaphoreType.DMA((12,)),
            pltpu.SemaphoreType.DMA((12,)),
            pltpu.SemaphoreType.DMA((6,)),
            pltpu.SemaphoreType.DMA((6,)),
            pltpu.SemaphoreType.DMA((3,)),
        ],
        compiler_params=pltpu.CompilerParams(collective_id=0),
    )(scale, partial)
    return out


# device time: 1206416 ns/iter; 2.0464x vs baseline; 1.0671x over previous
import jax
import jax.numpy as jnp
from jax import lax
from jax.experimental import pallas as pl
from jax.experimental.pallas import tpu as pltpu

N_DEV = 4
BM = 1024
MESH = pl.DeviceIdType.MESH


def kernel(x, w_mat, scale_x, scale_w):
    m, k = x.shape
    _, n = w_mat.shape
    hl = n // 2
    sc = hl // 2
    scale = (scale_x * scale_w).astype(jnp.float32)

    def body(scale_ref, x_ref, wm_ref, out_ref, rs_recv, w_ref, w0_ref,
             va, vb, rs_send_sems, rs_recv_sems, ag_send_sems,
             ag_recv_sems, cp_sems):
        my = lax.axis_index("i")
        left = lax.rem(my + (N_DEV - 1), N_DEV)
        right = lax.rem(my + 1, N_DEV)

        barrier = pltpu.get_barrier_semaphore()
        for nbr in (left, right):
            pl.semaphore_signal(barrier, inc=1, device_id=(nbr,),
                                device_id_type=MESH)
        pl.semaphore_wait(barrier, 2)

        def rows(b):
            return pl.ds(b * BM, BM)

        def cols(d, c):
            return pl.ds(d * hl + c * sc, sc)

        def mm(b, d, c):
            a16 = x_ref[rows(b), :].astype(jnp.bfloat16)
            b16 = wm_ref[:, cols(d, c)].astype(jnp.bfloat16)
            return lax.dot_general(
                a16, b16, (((1,), (0,)), ((), ())),
                preferred_element_type=jnp.float32,
            )

        def rs_start(s, d, c):
            tgt = right if d == 0 else left
            src = (w0_ref if s == 0 else w_ref).at[:, cols(d, c)]
            k_ = s * 4 + d * 2 + c
            r = pltpu.make_async_remote_copy(
                src_ref=src,
                dst_ref=rs_recv.at[s, :, cols(d, c)],
                send_sem=rs_send_sems.at[k_],
                recv_sem=rs_recv_sems.at[k_],
                device_id=(tgt,),
                device_id_type=MESH,
            )
            r.start()
            return r

        rdmas = {}
        for c, d in ((0, 0), (0, 1), (1, 0), (1, 1)):
            va[...] = mm(my, d, c)
            cpo = pltpu.make_async_copy(va, w0_ref.at[:, cols(d, c)],
                                        cp_sems.at[2])
            cpo.start()
            cpo.wait()
            rdmas[(0, d, c)] = rs_start(0, d, c)

        for s in range(3):
            bA = lax.rem(my + (2 * N_DEV - 1 - s), N_DEV)
            bB = lax.rem(my + (1 + s), N_DEV)
            for c, d in ((0, 0), (0, 1), (1, 0), (1, 1)):
                rdmas[(s, d, c)].wait()
                b = bA if d == 0 else bB
                cpb = pltpu.make_async_copy(rs_recv.at[s, :, cols(d, c)],
                                            vb, cp_sems.at[1])
                cpb.start()
                va[...] = mm(b, d, c)
                cpb.wait()
                if s < 2:
                    va[...] = va[...] + vb[...]
                    dst = w_ref.at[:, cols(d, c)]
                else:
                    va[...] = (va[...] + vb[...]) * scale_ref[0]
                    dst = out_ref.at[rows(b), cols(d, c)]
                cpo = pltpu.make_async_copy(va, dst, cp_sems.at[2])
                cpo.start()
                cpo.wait()
                if s < 2:
                    rdmas[(s + 1, d, c)] = rs_start(s + 1, d, c)

        def ag_start(s, d):
            tgt = right if d == 0 else left
            if d == 0:
                sb = lax.rem(my + (N_DEV + 1 - s), N_DEV)
            else:
                sb = lax.rem(my + (N_DEV - 1 + s), N_DEV)
            half = pl.ds(d * hl, hl)
            r = pltpu.make_async_remote_copy(
                src_ref=out_ref.at[rows(sb), half],
                dst_ref=out_ref.at[rows(sb), half],
                send_sem=ag_send_sems.at[s * 2 + d],
                recv_sem=ag_recv_sems.at[s * 2 + d],
                device_id=(tgt,),
                device_id_type=MESH,
            )
            r.start()
            return r

        ag = {(0, 0): ag_start(0, 0), (0, 1): ag_start(0, 1)}
        for s in range(3):
            for d in (0, 1):
                ag[(s, d)].wait()
                if s < 2:
                    ag[(s + 1, d)] = ag_start(s + 1, d)

    out, _, _, _ = pl.pallas_call(
        body,
        out_shape=[
            jax.ShapeDtypeStruct((m, n), jnp.float32),
            jax.ShapeDtypeStruct((3, BM, n), jnp.float32),
            jax.ShapeDtypeStruct((BM, n), jnp.float32),
            jax.ShapeDtypeStruct((BM, n), jnp.float32),
        ],
        in_specs=[
            pl.BlockSpec(memory_space=pltpu.MemorySpace.SMEM),
            pl.BlockSpec(memory_space=pltpu.MemorySpace.VMEM),
            pl.BlockSpec(memory_space=pltpu.MemorySpace.VMEM),
        ],
        out_specs=[
            pl.BlockSpec(memory_space=pl.ANY),
            pl.BlockSpec(memory_space=pl.ANY),
            pl.BlockSpec(memory_space=pl.ANY),
            pl.BlockSpec(memory_space=pl.ANY),
        ],
        scratch_shapes=[
            pltpu.VMEM((BM, sc), jnp.float32),
            pltpu.VMEM((BM, sc), jnp.float32),
            pltpu.SemaphoreType.DMA((12,)),
            pltpu.SemaphoreType.DMA((12,)),
            pltpu.SemaphoreType.DMA((6,)),
            pltpu.SemaphoreType.DMA((6,)),
            pltpu.SemaphoreType.DMA((3,)),
        ],
        compiler_params=pltpu.CompilerParams(collective_id=0),
    )(scale, x, w_mat)
    return out


# device time: 1191032 ns/iter; 2.0728x vs baseline; 1.0129x over previous
import jax
import jax.numpy as jnp
from jax import lax
from jax.experimental import pallas as pl
from jax.experimental.pallas import tpu as pltpu

N_DEV = 4
BM = 1024
MESH = pl.DeviceIdType.MESH


def kernel(x, w_mat, scale_x, scale_w):
    m, k = x.shape
    _, n = w_mat.shape
    hl = n // 2
    sc = hl // 2
    scale = (scale_x * scale_w).astype(jnp.float32)

    def body(scale_ref, x_ref, wm_ref, out_ref, rs_recv, w_ref, w0_ref,
             va, vb, rs_send_sems, rs_recv_sems, ag_send_sems,
             ag_recv_sems, cp_sems):
        my = lax.axis_index("i")
        left = lax.rem(my + (N_DEV - 1), N_DEV)
        right = lax.rem(my + 1, N_DEV)

        barrier = pltpu.get_barrier_semaphore()
        for nbr in (left, right):
            pl.semaphore_signal(barrier, inc=1, device_id=(nbr,),
                                device_id_type=MESH)
        pl.semaphore_wait(barrier, 2)

        def rows(b):
            return pl.ds(b * BM, BM)

        def cols(d, c):
            return pl.ds(d * hl + c * sc, sc)

        def mm(b, d, c):
            a16 = x_ref[rows(b), :].astype(jnp.bfloat16)
            b16 = wm_ref[:, cols(d, c)].astype(jnp.bfloat16)
            return lax.dot_general(
                a16, b16, (((1,), (0,)), ((), ())),
                preferred_element_type=jnp.float32,
            )

        def rs_start(s, d, c):
            tgt = right if d == 0 else left
            src = (w0_ref if s == 0 else w_ref).at[:, cols(d, c)]
            k_ = s * 4 + d * 2 + c
            r = pltpu.make_async_remote_copy(
                src_ref=src,
                dst_ref=rs_recv.at[s, :, cols(d, c)],
                send_sem=rs_send_sems.at[k_],
                recv_sem=rs_recv_sems.at[k_],
                device_id=(tgt,),
                device_id_type=MESH,
            )
            r.start()
            return r

        rdmas = {}
        for c, d in ((0, 0), (0, 1), (1, 0), (1, 1)):
            va[...] = mm(my, d, c)
            cpo = pltpu.make_async_copy(va, w0_ref.at[:, cols(d, c)],
                                        cp_sems.at[2])
            cpo.start()
            cpo.wait()
            rdmas[(0, d, c)] = rs_start(0, d, c)

        def ag_start(s, d, c):
            tgt = right if d == 0 else left
            if d == 0:
                sb = lax.rem(my + (N_DEV + 1 - s), N_DEV)
            else:
                sb = lax.rem(my + (N_DEV - 1 + s), N_DEV)
            k_ = s * 4 + d * 2 + c
            r = pltpu.make_async_remote_copy(
                src_ref=out_ref.at[rows(sb), cols(d, c)],
                dst_ref=out_ref.at[rows(sb), cols(d, c)],
                send_sem=ag_send_sems.at[k_],
                recv_sem=ag_recv_sems.at[k_],
                device_id=(tgt,),
                device_id_type=MESH,
            )
            r.start()
            return r

        ag = {}

        for s in range(3):
            bA = lax.rem(my + (2 * N_DEV - 1 - s), N_DEV)
            bB = lax.rem(my + (1 + s), N_DEV)
            for c, d in ((0, 0), (0, 1), (1, 0), (1, 1)):
                rdmas[(s, d, c)].wait()
                b = bA if d == 0 else bB
                cpb = pltpu.make_async_copy(rs_recv.at[s, :, cols(d, c)],
                                            vb, cp_sems.at[1])
                cpb.start()
                va[...] = mm(b, d, c)
                cpb.wait()
                if s < 2:
                    va[...] = va[...] + vb[...]
                    dst = w_ref.at[:, cols(d, c)]
                else:
                    va[...] = (va[...] + vb[...]) * scale_ref[0]
                    dst = out_ref.at[rows(b), cols(d, c)]
                cpo = pltpu.make_async_copy(va, dst, cp_sems.at[2])
                cpo.start()
                cpo.wait()
                if s < 2:
                    rdmas[(s + 1, d, c)] = rs_start(s + 1, d, c)
                else:
                    ag[(0, d, c)] = ag_start(0, d, c)

        for s in range(3):
            for c, d in ((0, 0), (0, 1), (1, 0), (1, 1)):
                ag[(s, d, c)].wait()
                if s < 2:
                    ag[(s + 1, d, c)] = ag_start(s + 1, d, c)

    out, _, _, _ = pl.pallas_call(
        body,
        out_shape=[
            jax.ShapeDtypeStruct((m, n), jnp.float32),
            jax.ShapeDtypeStruct((3, BM, n), jnp.float32),
            jax.ShapeDtypeStruct((BM, n), jnp.float32),
            jax.ShapeDtypeStruct((BM, n), jnp.float32),
        ],
        in_specs=[
            pl.BlockSpec(memory_space=pltpu.MemorySpace.SMEM),
            pl.BlockSpec(memory_space=pltpu.MemorySpace.VMEM),
            pl.BlockSpec(memory_space=pltpu.MemorySpace.VMEM),
        ],
        out_specs=[
            pl.BlockSpec(memory_space=pl.ANY),
            pl.BlockSpec(memory_space=pl.ANY),
            pl.BlockSpec(memory_space=pl.ANY),
            pl.BlockSpec(memory_space=pl.ANY),
        ],
        scratch_shapes=[
            pltpu.VMEM((BM, sc), jnp.float32),
            pltpu.VMEM((BM, sc), jnp.float32),
            pltpu.SemaphoreType.DMA((12,)),
            pltpu.SemaphoreType.DMA((12,)),
            pltpu.SemaphoreType.DMA((12,)),
            pltpu.SemaphoreType.DMA((12,)),
            pltpu.SemaphoreType.DMA((3,)),
        ],
        compiler_params=pltpu.CompilerParams(collective_id=0),
    )(scale, x, w_mat)
    return out


# device time: 1184393 ns/iter; 2.0844x vs baseline; 1.0056x over previous
import jax
import jax.numpy as jnp
from jax import lax
from jax.experimental import pallas as pl
from jax.experimental.pallas import tpu as pltpu

N_DEV = 4
BM = 1024
NC = 4
MESH = pl.DeviceIdType.MESH


def kernel(x, w_mat, scale_x, scale_w):
    m, k = x.shape
    _, n = w_mat.shape
    hl = n // 2
    sc = hl // NC
    scale = (scale_x * scale_w).astype(jnp.float32)

    def body(scale_ref, x_ref, wm_ref, out_ref, rs_recv, w_ref, w0_ref,
             va, vb, rs_send_sems, rs_recv_sems, ag_send_sems,
             ag_recv_sems, cp_sems):
        my = lax.axis_index("i")
        left = lax.rem(my + (N_DEV - 1), N_DEV)
        right = lax.rem(my + 1, N_DEV)

        barrier = pltpu.get_barrier_semaphore()
        for nbr in (left, right):
            pl.semaphore_signal(barrier, inc=1, device_id=(nbr,),
                                device_id_type=MESH)
        pl.semaphore_wait(barrier, 2)

        def rows(b):
            return pl.ds(b * BM, BM)

        def cols(d, c):
            return pl.ds(d * hl + c * sc, sc)

        def mm(b, d, c):
            a16 = x_ref[rows(b), :].astype(jnp.bfloat16)
            b16 = wm_ref[:, cols(d, c)].astype(jnp.bfloat16)
            return lax.dot_general(
                a16, b16, (((1,), (0,)), ((), ())),
                preferred_element_type=jnp.float32,
            )

        def rs_start(s, d, c):
            tgt = right if d == 0 else left
            src = (w0_ref if s == 0 else w_ref).at[:, cols(d, c)]
            k_ = (s * 2 + d) * NC + c
            r = pltpu.make_async_remote_copy(
                src_ref=src,
                dst_ref=rs_recv.at[s, :, cols(d, c)],
                send_sem=rs_send_sems.at[k_],
                recv_sem=rs_recv_sems.at[k_],
                device_id=(tgt,),
                device_id_type=MESH,
            )
            r.start()
            return r

        rdmas = {}
        for c in range(NC):
          for d in (0, 1):
            va[...] = mm(my, d, c)
            cpo = pltpu.make_async_copy(va, w0_ref.at[:, cols(d, c)],
                                        cp_sems.at[2])
            cpo.start()
            cpo.wait()
            rdmas[(0, d, c)] = rs_start(0, d, c)

        def ag_start(s, d, c):
            tgt = right if d == 0 else left
            if d == 0:
                sb = lax.rem(my + (N_DEV + 1 - s), N_DEV)
            else:
                sb = lax.rem(my + (N_DEV - 1 + s), N_DEV)
            k_ = (s * 2 + d) * NC + c
            r = pltpu.make_async_remote_copy(
                src_ref=out_ref.at[rows(sb), cols(d, c)],
                dst_ref=out_ref.at[rows(sb), cols(d, c)],
                send_sem=ag_send_sems.at[k_],
                recv_sem=ag_recv_sems.at[k_],
                device_id=(tgt,),
                device_id_type=MESH,
            )
            r.start()
            return r

        ag = {}

        for s in range(3):
            bA = lax.rem(my + (2 * N_DEV - 1 - s), N_DEV)
            bB = lax.rem(my + (1 + s), N_DEV)
            for c in range(NC):
              for d in (0, 1):
                rdmas[(s, d, c)].wait()
                b = bA if d == 0 else bB
                cpb = pltpu.make_async_copy(rs_recv.at[s, :, cols(d, c)],
                                            vb, cp_sems.at[1])
                cpb.start()
                va[...] = mm(b, d, c)
                cpb.wait()
                if s < 2:
                    va[...] = va[...] + vb[...]
                    dst = w_ref.at[:, cols(d, c)]
                else:
                    va[...] = (va[...] + vb[...]) * scale_ref[0]
                    dst = out_ref.at[rows(b), cols(d, c)]
                cpo = pltpu.make_async_copy(va, dst, cp_sems.at[2])
                cpo.start()
                cpo.wait()
                if s < 2:
                    rdmas[(s + 1, d, c)] = rs_start(s + 1, d, c)
                else:
                    ag[(0, d, c)] = ag_start(0, d, c)

        for s in range(3):
            for c in range(NC):
              for d in (0, 1):
                ag[(s, d, c)].wait()
                if s < 2:
                    ag[(s + 1, d, c)] = ag_start(s + 1, d, c)

    out, _, _, _ = pl.pallas_call(
        body,
        out_shape=[
            jax.ShapeDtypeStruct((m, n), jnp.float32),
            jax.ShapeDtypeStruct((3, BM, n), jnp.float32),
            jax.ShapeDtypeStruct((BM, n), jnp.float32),
            jax.ShapeDtypeStruct((BM, n), jnp.float32),
        ],
        in_specs=[
            pl.BlockSpec(memory_space=pltpu.MemorySpace.SMEM),
            pl.BlockSpec(memory_space=pltpu.MemorySpace.VMEM),
            pl.BlockSpec(memory_space=pltpu.MemorySpace.VMEM),
        ],
        out_specs=[
            pl.BlockSpec(memory_space=pl.ANY),
            pl.BlockSpec(memory_space=pl.ANY),
            pl.BlockSpec(memory_space=pl.ANY),
            pl.BlockSpec(memory_space=pl.ANY),
        ],
        scratch_shapes=[
            pltpu.VMEM((BM, sc), jnp.float32),
            pltpu.VMEM((BM, sc), jnp.float32),
            pltpu.SemaphoreType.DMA((6 * NC,)),
            pltpu.SemaphoreType.DMA((6 * NC,)),
            pltpu.SemaphoreType.DMA((6 * NC,)),
            pltpu.SemaphoreType.DMA((6 * NC,)),
            pltpu.SemaphoreType.DMA((3,)),
        ],
        compiler_params=pltpu.CompilerParams(collective_id=0),
    )(scale, x, w_mat)
    return out


# device time: 647670 ns/iter; 3.8118x vs baseline; 1.8287x over previous
import jax
import jax.numpy as jnp
from jax import lax
from jax.experimental import pallas as pl
from jax.experimental.pallas import tpu as pltpu

N_DEV = 4
BM = 1024
NC = 4
MESH = pl.DeviceIdType.MESH


def kernel(x, w_mat, scale_x, scale_w):
    m, k = x.shape
    _, n = w_mat.shape
    hl = n // 2
    sc = hl // NC
    scale = (scale_x * scale_w).astype(jnp.float32)

    def body(scale_ref, x_ref, wm_ref, out_ref, rs_recv, w_ref, w0_ref,
             agd, va, vb16, vc16, rs_send_sems, rs_recv_sems,
             ag_send_sems, ag_recv_sems, cp_sems):
        my = lax.axis_index("i")
        left = lax.rem(my + (N_DEV - 1), N_DEV)
        right = lax.rem(my + 1, N_DEV)

        barrier = pltpu.get_barrier_semaphore()
        for nbr in (left, right):
            pl.semaphore_signal(barrier, inc=1, device_id=(nbr,),
                                device_id_type=MESH)
        pl.semaphore_wait(barrier, 2)

        def rows(b):
            return pl.ds(b * BM, BM)

        def cols(d, c):
            return pl.ds(d * hl + c * sc, sc)

        def mm(b, d, c):
            a16 = x_ref[rows(b), :].astype(jnp.bfloat16)
            b16 = wm_ref[:, cols(d, c)].astype(jnp.bfloat16)
            return lax.dot_general(
                a16, b16, (((1,), (0,)), ((), ())),
                preferred_element_type=jnp.float32,
            )

        def rs_start(s, d, c):
            tgt = right if d == 0 else left
            src = (w0_ref if s == 0 else w_ref).at[:, cols(d, c)]
            k_ = (s * 2 + d) * NC + c
            r = pltpu.make_async_remote_copy(
                src_ref=src,
                dst_ref=rs_recv.at[s, :, cols(d, c)],
                send_sem=rs_send_sems.at[k_],
                recv_sem=rs_recv_sems.at[k_],
                device_id=(tgt,),
                device_id_type=MESH,
            )
            r.start()
            return r

        rdmas = {}
        for c in range(NC):
          for d in (0, 1):
            va[...] = mm(my, d, c)
            vc16[...] = va[...].astype(jnp.bfloat16)
            cpo = pltpu.make_async_copy(vc16, w0_ref.at[:, cols(d, c)],
                                        cp_sems.at[2])
            cpo.start()
            cpo.wait()
            rdmas[(0, d, c)] = rs_start(0, d, c)

        def ag_start(s, d, c):
            tgt = right if d == 0 else left
            if d == 0:
                sb = lax.rem(my + (N_DEV + 1 - s), N_DEV)
            else:
                sb = lax.rem(my + (N_DEV - 1 + s), N_DEV)
            del sb
            k_ = (s * 2 + d) * NC + c
            r = pltpu.make_async_remote_copy(
                src_ref=agd.at[s, :, cols(d, c)],
                dst_ref=agd.at[s + 1, :, cols(d, c)],
                send_sem=ag_send_sems.at[k_],
                recv_sem=ag_recv_sems.at[k_],
                device_id=(tgt,),
                device_id_type=MESH,
            )
            r.start()
            return r

        ag = {}

        for s in range(3):
            bA = lax.rem(my + (2 * N_DEV - 1 - s), N_DEV)
            bB = lax.rem(my + (1 + s), N_DEV)
            for c in range(NC):
              for d in (0, 1):
                rdmas[(s, d, c)].wait()
                b = bA if d == 0 else bB
                cpb = pltpu.make_async_copy(rs_recv.at[s, :, cols(d, c)],
                                            vb16, cp_sems.at[1])
                cpb.start()
                va[...] = mm(b, d, c)
                cpb.wait()
                if s < 2:
                    vc16[...] = (va[...] + vb16[...].astype(jnp.float32)
                                 ).astype(jnp.bfloat16)
                    cpo = pltpu.make_async_copy(
                        vc16, w_ref.at[:, cols(d, c)], cp_sems.at[2])
                    cpo.start()
                    cpo.wait()
                    rdmas[(s + 1, d, c)] = rs_start(s + 1, d, c)
                else:
                    va[...] = (va[...] + vb16[...].astype(jnp.float32)
                               ) * scale_ref[0]
                    vc16[...] = va[...].astype(jnp.bfloat16)
                    cpo = pltpu.make_async_copy(
                        va, out_ref.at[rows(b), cols(d, c)], cp_sems.at[2])
                    cpa = pltpu.make_async_copy(
                        vc16, agd.at[0, :, cols(d, c)], cp_sems.at[0])
                    cpo.start()
                    cpa.start()
                    cpo.wait()
                    cpa.wait()
                    ag[(0, d, c)] = ag_start(0, d, c)

        for s in range(3):
            gA = lax.rem(my + (N_DEV - s), N_DEV)
            gB = lax.rem(my + s, N_DEV)
            for c in range(NC):
              for d in (0, 1):
                ag[(s, d, c)].wait()
                if s < 2:
                    ag[(s + 1, d, c)] = ag_start(s + 1, d, c)
                g = gA if d == 0 else gB
                cpb = pltpu.make_async_copy(agd.at[s + 1, :, cols(d, c)],
                                            vb16, cp_sems.at[1])
                cpb.start()
                cpb.wait()
                va[...] = vb16[...].astype(jnp.float32)
                cpo = pltpu.make_async_copy(
                    va, out_ref.at[rows(g), cols(d, c)], cp_sems.at[2])
                cpo.start()
                cpo.wait()

    out, _, _, _, _ = pl.pallas_call(
        body,
        out_shape=[
            jax.ShapeDtypeStruct((m, n), jnp.float32),
            jax.ShapeDtypeStruct((3, BM, n), jnp.bfloat16),
            jax.ShapeDtypeStruct((BM, n), jnp.bfloat16),
            jax.ShapeDtypeStruct((BM, n), jnp.bfloat16),
            jax.ShapeDtypeStruct((4, BM, n), jnp.bfloat16),
        ],
        in_specs=[
            pl.BlockSpec(memory_space=pltpu.MemorySpace.SMEM),
            pl.BlockSpec(memory_space=pltpu.MemorySpace.VMEM),
            pl.BlockSpec(memory_space=pltpu.MemorySpace.VMEM),
        ],
        out_specs=[
            pl.BlockSpec(memory_space=pl.ANY),
            pl.BlockSpec(memory_space=pl.ANY),
            pl.BlockSpec(memory_space=pl.ANY),
            pl.BlockSpec(memory_space=pl.ANY),
            pl.BlockSpec(memory_space=pl.ANY),
        ],
        scratch_shapes=[
            pltpu.VMEM((BM, sc), jnp.float32),
            pltpu.VMEM((BM, sc), jnp.bfloat16),
            pltpu.VMEM((BM, sc), jnp.bfloat16),
            pltpu.SemaphoreType.DMA((6 * NC,)),
            pltpu.SemaphoreType.DMA((6 * NC,)),
            pltpu.SemaphoreType.DMA((6 * NC,)),
            pltpu.SemaphoreType.DMA((6 * NC,)),
            pltpu.SemaphoreType.DMA((3,)),
        ],
        compiler_params=pltpu.CompilerParams(collective_id=0),
    )(scale, x, w_mat)
    return out
